# baseline (speedup 1.0000x reference)
import functools

import jax
import jax.numpy as jnp
from jax.experimental import pallas as pl
from jax.experimental.pallas import tpu as pltpu

E = 8
TOPK = 2
NEG = -1e30


def _router_body(x_ref, wgt_ref, logits_ref, w_ref, i0_ref, i1_ref,
                 w0_ref, w1_ref, c0_ref, call_ref, psum_ref, bl_ref):
    i = pl.program_id(0)
    nsteps = pl.num_programs(0)
    tb = x_ref.shape[0]

    lp = jnp.dot(x_ref[...], wgt_ref[...], preferred_element_type=jnp.float32)
    lanes = jax.lax.broadcasted_iota(jnp.int32, lp.shape, 1)
    valid = lanes < E
    l = jnp.where(valid, lp, NEG)

    m0 = jnp.max(l, axis=1, keepdims=True)
    i0 = jnp.min(jnp.where(l == m0, lanes, 127), axis=1, keepdims=True)
    l2 = jnp.where(lanes == i0, NEG, l)
    m1 = jnp.max(l2, axis=1, keepdims=True)
    i1 = jnp.min(jnp.where(l2 == m1, lanes, 127), axis=1, keepdims=True)

    w0 = jax.nn.sigmoid(m0 - m1)
    w1 = 1.0 - w0

    oh0 = (lanes == i0).astype(jnp.float32)
    oh1 = (lanes == i1).astype(jnp.float32)

    logits_ref[...] = lp[:, :E]
    w_ref[...] = (w0 * oh0 + w1 * oh1)[:, :E]
    i0_ref[...] = i0
    i1_ref[...] = i1
    w0_ref[...] = w0
    w1_ref[...] = w1

    p = jnp.where(valid, jnp.exp(l - m0), 0.0)
    p = p / jnp.sum(p, axis=1, keepdims=True)

    c0_part = jnp.sum(oh0, axis=0, keepdims=True)
    call_part = c0_part + jnp.sum(oh1, axis=0, keepdims=True)
    psum_part = jnp.sum(p, axis=0, keepdims=True)

    @pl.when(i == 0)
    def _init():
        c0_ref[...] = c0_part
        call_ref[...] = call_part
        psum_ref[...] = psum_part

    @pl.when(i > 0)
    def _acc():
        c0_ref[...] += c0_part
        call_ref[...] += call_part
        psum_ref[...] += psum_part

    @pl.when(i == nsteps - 1)
    def _fin():
        t_total = jnp.float32(nsteps * tb)
        bl = (jnp.float32(E) / (t_total * t_total)) * jnp.sum(
            call_ref[...] * psum_ref[...])
        bl_ref[...] = jnp.reshape(bl, (1, 1))


def _run_router(x, Wg):
    t, d = x.shape
    tb = 512 if t % 512 == 0 else t
    wgt = jnp.zeros((d, 128), jnp.float32).at[:, :E].set(Wg.T.astype(jnp.float32))
    grid = (t // tb,)
    outs = pl.pallas_call(
        _router_body,
        grid=grid,
        in_specs=[
            pl.BlockSpec((tb, d), lambda i: (i, 0)),
            pl.BlockSpec((d, 128), lambda i: (0, 0)),
        ],
        out_specs=[
            pl.BlockSpec((tb, E), lambda i: (i, 0)),
            pl.BlockSpec((tb, E), lambda i: (i, 0)),
            pl.BlockSpec((tb, 1), lambda i: (i, 0)),
            pl.BlockSpec((tb, 1), lambda i: (i, 0)),
            pl.BlockSpec((tb, 1), lambda i: (i, 0)),
            pl.BlockSpec((tb, 1), lambda i: (i, 0)),
            pl.BlockSpec((1, 128), lambda i: (0, 0)),
            pl.BlockSpec((1, 128), lambda i: (0, 0)),
            pl.BlockSpec((1, 128), lambda i: (0, 0)),
            pl.BlockSpec((1, 1), lambda i: (0, 0)),
        ],
        out_shape=[
            jax.ShapeDtypeStruct((t, E), jnp.float32),
            jax.ShapeDtypeStruct((t, E), jnp.float32),
            jax.ShapeDtypeStruct((t, 1), jnp.int32),
            jax.ShapeDtypeStruct((t, 1), jnp.int32),
            jax.ShapeDtypeStruct((t, 1), jnp.float32),
            jax.ShapeDtypeStruct((t, 1), jnp.float32),
            jax.ShapeDtypeStruct((1, 128), jnp.float32),
            jax.ShapeDtypeStruct((1, 128), jnp.float32),
            jax.ShapeDtypeStruct((1, 128), jnp.float32),
            jax.ShapeDtypeStruct((1, 1), jnp.float32),
        ],
    )(x, wgt)
    return outs


def _dense_body(x_ref, gw_ref, pw_ref, ow_ref, w_ref, out_ref):
    e = pl.program_id(1)
    f = pl.program_id(2)

    lanes = jax.lax.broadcasted_iota(jnp.int32, w_ref.shape, 1)
    wcol = jnp.sum(jnp.where(lanes == e, w_ref[...], 0.0), axis=1, keepdims=True)

    g = jnp.dot(x_ref[...], gw_ref[0], preferred_element_type=jnp.float32)
    p = jnp.dot(x_ref[...], pw_ref[0], preferred_element_type=jnp.float32)
    h = (g * (p * jax.nn.sigmoid(p))) * wcol
    part = jnp.dot(h.astype(ow_ref.dtype), ow_ref[0],
                   preferred_element_type=jnp.float32)

    @pl.when((e == 0) & (f == 0))
    def _init():
        out_ref[...] = part

    @pl.when((e > 0) | (f > 0))
    def _acc():
        out_ref[...] += part


def kernel(hidden_states, Wg, gw, pw, ow):
    b, s, d = hidden_states.shape
    x = hidden_states.reshape(-1, d).astype(jnp.float32)
    t = x.shape[0]
    ne, _, fdim = gw.shape

    (logits, wdense, _i0, _i1, _w0, _w1, _c0, _call, _psum, bl) = _run_router(x, Wg)

    tb = 512 if t % 512 == 0 else t
    fb = 512 if fdim % 512 == 0 else fdim
    grid = (t // tb, ne, fdim // fb)

    out = pl.pallas_call(
        _dense_body,
        grid=grid,
        in_specs=[
            pl.BlockSpec((tb, d), lambda i, e, f: (i, 0)),
            pl.BlockSpec((1, d, fb), lambda i, e, f: (e, 0, f)),
            pl.BlockSpec((1, d, fb), lambda i, e, f: (e, 0, f)),
            pl.BlockSpec((1, fb, d), lambda i, e, f: (e, f, 0)),
            pl.BlockSpec((tb, E), lambda i, e, f: (i, 0)),
        ],
        out_specs=pl.BlockSpec((tb, d), lambda i, e, f: (i, 0)),
        out_shape=jax.ShapeDtypeStruct((t, d), jnp.float32),
    )(x.astype(jnp.bfloat16), gw.astype(jnp.bfloat16), pw.astype(jnp.bfloat16),
      ow.astype(jnp.bfloat16), wdense)

    return (out.reshape(b, s, d), logits, bl[0, 0])


# speedup vs baseline: 1.1717x; 1.1717x over previous
import functools

import jax
import jax.numpy as jnp
from jax import lax
from jax.experimental import pallas as pl
from jax.experimental.pallas import tpu as pltpu
from jax.experimental.pallas import tpu_sc as plsc

E = 8
TOPK = 2
NEG = -1e30
BLK = 512


def _router_body(x_ref, wgt_ref, logits_ref, i0_ref, i1_ref,
                 w0_ref, w1_ref, c0_ref, call_ref, psum_ref, bl_ref):
    i = pl.program_id(0)
    nsteps = pl.num_programs(0)
    tb = x_ref.shape[0]

    lp = jnp.dot(x_ref[...], wgt_ref[...], preferred_element_type=jnp.float32)
    lanes = jax.lax.broadcasted_iota(jnp.int32, lp.shape, 1)
    valid = lanes < E
    l = jnp.where(valid, lp, NEG)

    m0 = jnp.max(l, axis=1, keepdims=True)
    i0 = jnp.min(jnp.where(l == m0, lanes, 127), axis=1, keepdims=True)
    l2 = jnp.where(lanes == i0, NEG, l)
    m1 = jnp.max(l2, axis=1, keepdims=True)
    i1 = jnp.min(jnp.where(l2 == m1, lanes, 127), axis=1, keepdims=True)

    w0 = jax.nn.sigmoid(m0 - m1)

    oh0 = (lanes == i0).astype(jnp.float32)
    oh1 = (lanes == i1).astype(jnp.float32)

    logits_ref[...] = lp[:, :E]
    i0_ref[...] = i0
    i1_ref[...] = i1
    w0_ref[...] = w0
    w1_ref[...] = 1.0 - w0

    p = jnp.where(valid, jnp.exp(l - m0), 0.0)
    p = p / jnp.sum(p, axis=1, keepdims=True)

    c0_part = jnp.sum(oh0, axis=0, keepdims=True)
    call_part = c0_part + jnp.sum(oh1, axis=0, keepdims=True)
    psum_part = jnp.sum(p, axis=0, keepdims=True)

    @pl.when(i == 0)
    def _init():
        c0_ref[...] = c0_part
        call_ref[...] = call_part
        psum_ref[...] = psum_part

    @pl.when(i > 0)
    def _acc():
        c0_ref[...] += c0_part
        call_ref[...] += call_part
        psum_ref[...] += psum_part

    @pl.when(i == nsteps - 1)
    def _fin():
        t_total = jnp.float32(nsteps * tb)
        bl = (jnp.float32(E) / (t_total * t_total)) * jnp.sum(
            call_ref[...] * psum_ref[...])
        bl_ref[...] = jnp.reshape(bl, (1, 1))


def _run_router(x, Wg):
    t, d = x.shape
    tb = 512 if t % 512 == 0 else t
    wgt = jnp.zeros((d, 128), jnp.float32).at[:, :E].set(Wg.T.astype(jnp.float32))
    return pl.pallas_call(
        _router_body,
        grid=(t // tb,),
        in_specs=[
            pl.BlockSpec((tb, d), lambda i: (i, 0)),
            pl.BlockSpec((d, 128), lambda i: (0, 0)),
        ],
        out_specs=[
            pl.BlockSpec((tb, E), lambda i: (i, 0)),
            pl.BlockSpec((tb, 1), lambda i: (i, 0)),
            pl.BlockSpec((tb, 1), lambda i: (i, 0)),
            pl.BlockSpec((tb, 1), lambda i: (i, 0)),
            pl.BlockSpec((tb, 1), lambda i: (i, 0)),
            pl.BlockSpec((1, 128), lambda i: (0, 0)),
            pl.BlockSpec((1, 128), lambda i: (0, 0)),
            pl.BlockSpec((1, 128), lambda i: (0, 0)),
            pl.BlockSpec((1, 1), lambda i: (0, 0)),
        ],
        out_shape=[
            jax.ShapeDtypeStruct((t, E), jnp.float32),
            jax.ShapeDtypeStruct((t, 1), jnp.int32),
            jax.ShapeDtypeStruct((t, 1), jnp.int32),
            jax.ShapeDtypeStruct((t, 1), jnp.float32),
            jax.ShapeDtypeStruct((t, 1), jnp.float32),
            jax.ShapeDtypeStruct((1, 128), jnp.float32),
            jax.ShapeDtypeStruct((1, 128), jnp.float32),
            jax.ShapeDtypeStruct((1, 128), jnp.float32),
            jax.ShapeDtypeStruct((1, 1), jnp.float32),
        ],
    )(x, wgt)


def _rank_body(i0_ref, i1_ref, off0_ref, off1_ref, p0_ref, p1_ref,
               acc0_ref, acc1_ref, tri_ref):
    i = pl.program_id(0)
    tb = i0_ref.shape[0]

    @pl.when(i == 0)
    def _init():
        r = jax.lax.broadcasted_iota(jnp.int32, (tb, tb), 0)
        c = jax.lax.broadcasted_iota(jnp.int32, (tb, tb), 1)
        tri_ref[...] = (c <= r).astype(jnp.float32)
        acc0_ref[...] = jnp.zeros_like(acc0_ref)
        acc1_ref[...] = jnp.zeros_like(acc1_ref)

    lanes = jax.lax.broadcasted_iota(jnp.int32, (tb, 128), 1)
    oh0 = (lanes == i0_ref[...]).astype(jnp.float32)
    oh1 = (lanes == i1_ref[...]).astype(jnp.float32)
    cum0 = jnp.dot(tri_ref[...], oh0, preferred_element_type=jnp.float32)
    cum1 = jnp.dot(tri_ref[...], oh1, preferred_element_type=jnp.float32)

    pos0 = jnp.sum(oh0 * (off0_ref[...] + acc0_ref[...] + cum0 - 1.0),
                   axis=1, keepdims=True)
    pos1 = jnp.sum(oh1 * (off1_ref[...] + acc1_ref[...] + cum1 - 1.0),
                   axis=1, keepdims=True)
    p0_ref[...] = pos0.astype(jnp.int32)
    p1_ref[...] = pos1.astype(jnp.int32)

    acc0_ref[...] += jnp.sum(oh0, axis=0, keepdims=True)
    acc1_ref[...] += jnp.sum(oh1, axis=0, keepdims=True)


def _run_rank(i0, i1, off0, off1):
    t = i0.shape[0]
    tb = 1024 if t % 1024 == 0 else t
    return pl.pallas_call(
        _rank_body,
        grid=(t // tb,),
        in_specs=[
            pl.BlockSpec((tb, 1), lambda i: (i, 0)),
            pl.BlockSpec((tb, 1), lambda i: (i, 0)),
            pl.BlockSpec((1, 128), lambda i: (0, 0)),
            pl.BlockSpec((1, 128), lambda i: (0, 0)),
        ],
        out_specs=[
            pl.BlockSpec((tb, 1), lambda i: (i, 0)),
            pl.BlockSpec((tb, 1), lambda i: (i, 0)),
        ],
        out_shape=[
            jax.ShapeDtypeStruct((t, 1), jnp.int32),
            jax.ShapeDtypeStruct((t, 1), jnp.int32),
        ],
        scratch_shapes=[
            pltpu.VMEM((1, 128), jnp.float32),
            pltpu.VMEM((1, 128), jnp.float32),
            pltpu.VMEM((tb, tb), jnp.float32),
        ],
    )(i0, i1, off0, off1)


def _sc_mesh():
    return plsc.VectorSubcoreMesh(core_axis_name="c", subcore_axis_name="s")


def _run_scatter(p0, p1, tok, w0, w1, nslot):
    t = p0.shape[0]
    info = plsc.get_sparse_core_info()
    nw = info.num_cores * info.num_subcores
    chunk = t // nw

    @functools.partial(
        pl.kernel,
        out_type=[
            jax.ShapeDtypeStruct((nslot,), jnp.int32),
            jax.ShapeDtypeStruct((nslot,), jnp.float32),
        ],
        mesh=_sc_mesh(),
        scratch_types=[
            pltpu.VMEM((chunk,), jnp.int32),
            pltpu.VMEM((chunk,), jnp.int32),
            pltpu.VMEM((chunk,), jnp.float32),
            pltpu.SemaphoreType.DMA,
        ],
    )
    def run(p0_h, p1_h, tok_h, w0_h, w1_h, src_h, wsl_h, idx_v, tok_v, w_v, sem):
        wid = lax.axis_index("s") * info.num_cores + lax.axis_index("c")
        base = wid * chunk
        pltpu.sync_copy(tok_h.at[pl.ds(base, chunk)], tok_v)
        pltpu.sync_copy(p0_h.at[pl.ds(base, chunk)], idx_v)
        pltpu.async_copy(tok_v, src_h.at[idx_v], sem).wait()
        pltpu.sync_copy(w0_h.at[pl.ds(base, chunk)], w_v)
        pltpu.async_copy(w_v, wsl_h.at[idx_v], sem).wait()
        pltpu.sync_copy(p1_h.at[pl.ds(base, chunk)], idx_v)
        pltpu.async_copy(tok_v, src_h.at[idx_v], sem).wait()
        pltpu.sync_copy(w1_h.at[pl.ds(base, chunk)], w_v)
        pltpu.async_copy(w_v, wsl_h.at[idx_v], sem).wait()

    return run(p0, p1, tok, w0, w1)


def _run_gather(table, idx, clamp_hi, chunk_rows=16):
    n = idx.shape[0]
    d = table.shape[1]
    info = plsc.get_sparse_core_info()
    nw = info.num_cores * info.num_subcores
    per_w = n // nw
    nch = per_w // chunk_rows

    @functools.partial(
        pl.kernel,
        out_type=jax.ShapeDtypeStruct((n, d), table.dtype),
        mesh=_sc_mesh(),
        scratch_types=[
            pltpu.VMEM((per_w,), jnp.int32),
            pltpu.VMEM((chunk_rows, d), table.dtype),
            pltpu.VMEM((chunk_rows, d), table.dtype),
            pltpu.SemaphoreType.DMA,
            pltpu.SemaphoreType.DMA,
        ],
    )
    def run(tab_h, idx_h, out_h, idx_v, buf0, buf1, sem0, sem1):
        wid = lax.axis_index("s") * info.num_cores + lax.axis_index("c")
        base = wid * per_w
        pltpu.sync_copy(idx_h.at[pl.ds(base, per_w)], idx_v)
        if clamp_hi is not None:
            for j in range(per_w // 16):
                sl = pl.ds(j * 16, 16)
                v = idx_v[sl]
                idx_v[sl] = jnp.minimum(jnp.maximum(v, 0), clamp_hi)
        bufs = (buf0, buf1)
        sems = (sem0, sem1)
        cps = []
        for c in range(nch):
            b = bufs[c % 2]
            cp = pltpu.async_copy(
                tab_h.at[idx_v.at[pl.ds(c * chunk_rows, chunk_rows)]],
                b, sems[c % 2])
            cps.append(cp)
            if c >= 1:
                cps[c - 1].wait()
                pltpu.sync_copy(bufs[(c - 1) % 2],
                                out_h.at[pl.ds(base + (c - 1) * chunk_rows,
                                               chunk_rows)])
        cps[nch - 1].wait()
        pltpu.sync_copy(bufs[(nch - 1) % 2],
                        out_h.at[pl.ds(base + (nch - 1) * chunk_rows,
                                       chunk_rows)])

    return run(table, idx)


def _run_gather2(table, idx0, idx1):
    ya = _run_gather(table, idx0, None)
    yb = _run_gather(table, idx1, None)
    return ya, yb


def _mlp_body(be_ref, xs_ref, gw_ref, pw_ref, ow_ref, wsl_ref, ys_ref):
    xb = xs_ref[...].astype(jnp.bfloat16)
    g = jnp.dot(xb, gw_ref[0], preferred_element_type=jnp.float32)
    p = jnp.dot(xb, pw_ref[0], preferred_element_type=jnp.float32)
    h = (g * (p * jax.nn.sigmoid(p))) * wsl_ref[...]
    ys_ref[...] = jnp.dot(h.astype(jnp.bfloat16), ow_ref[0],
                          preferred_element_type=jnp.float32)


def _run_mlp(xs, gwb, pwb, owb, wslot, block_expert):
    nslot, d = xs.shape
    fdim = gwb.shape[2]
    nblk = nslot // BLK
    grid_spec = pltpu.PrefetchScalarGridSpec(
        num_scalar_prefetch=1,
        grid=(nblk,),
        in_specs=[
            pl.BlockSpec((BLK, d), lambda b, be: (b, 0)),
            pl.BlockSpec((1, d, fdim), lambda b, be: (be[b], 0, 0)),
            pl.BlockSpec((1, d, fdim), lambda b, be: (be[b], 0, 0)),
            pl.BlockSpec((1, fdim, d), lambda b, be: (be[b], 0, 0)),
            pl.BlockSpec((BLK, 1), lambda b, be: (b, 0)),
        ],
        out_specs=pl.BlockSpec((BLK, d), lambda b, be: (b, 0)),
    )
    return pl.pallas_call(
        _mlp_body,
        grid_spec=grid_spec,
        out_shape=jax.ShapeDtypeStruct((nslot, d), jnp.float32),
    )(block_expert, xs, gwb, pwb, owb, wslot)


def _combine_body(ya_ref, yb_ref, out_ref):
    out_ref[...] = ya_ref[...] + yb_ref[...]


def _run_combine(ya, yb):
    t, d = ya.shape
    tb = 512 if t % 512 == 0 else t
    return pl.pallas_call(
        _combine_body,
        grid=(t // tb,),
        in_specs=[
            pl.BlockSpec((tb, d), lambda i: (i, 0)),
            pl.BlockSpec((tb, d), lambda i: (i, 0)),
        ],
        out_specs=pl.BlockSpec((tb, d), lambda i: (i, 0)),
        out_shape=jax.ShapeDtypeStruct((t, d), jnp.float32),
    )(ya, yb)


def kernel(hidden_states, Wg, gw, pw, ow):
    b, s, d = hidden_states.shape
    x = hidden_states.reshape(-1, d).astype(jnp.float32)
    t = x.shape[0]
    ne, _, fdim = gw.shape
    nslot = TOPK * t + ne * BLK

    (logits, i0, i1, w0, w1, c0, call, _psum, bl) = _run_router(x, Wg)

    c0v = c0[0, :ne].astype(jnp.int32)
    callv = call[0, :ne].astype(jnp.int32)
    padded = ((callv + (BLK - 1)) // BLK) * BLK
    base = jnp.cumsum(padded) - padded
    off0 = jnp.zeros((1, 128), jnp.float32).at[0, :ne].set(base.astype(jnp.float32))
    off1 = jnp.zeros((1, 128), jnp.float32).at[0, :ne].set(
        (base + c0v).astype(jnp.float32))
    ends = (base + padded) // BLK
    nblk = nslot // BLK
    block_expert = jnp.minimum(
        jnp.sum(jnp.arange(nblk)[:, None] >= ends[None, :], axis=1),
        ne - 1).astype(jnp.int32)

    p0, p1 = _run_rank(i0, i1, off0, off1)
    p0f, p1f = p0.reshape(t), p1.reshape(t)

    tok = jax.lax.iota(jnp.int32, t)
    src, wslot = _run_scatter(p0f, p1f, tok, w0.reshape(t), w1.reshape(t), nslot)

    xs = _run_gather(x, src, t - 1)
    ys = _run_mlp(xs, gw.astype(jnp.bfloat16), pw.astype(jnp.bfloat16),
                  ow.astype(jnp.bfloat16), wslot.reshape(nslot, 1),
                  block_expert)
    ya, yb = _run_gather2(ys, p0f, p1f)
    out = _run_combine(ya, yb)

    return (out.reshape(b, s, d), logits, bl[0, 0])


# speedup vs baseline: 1.7596x; 1.5017x over previous
import functools

import jax
import jax.numpy as jnp
from jax import lax
from jax.experimental import pallas as pl
from jax.experimental.pallas import tpu as pltpu
from jax.experimental.pallas import tpu_sc as plsc

E = 8
TOPK = 2
NEG = -1e30
BLK = 512


def _router_body(x_ref, wgt_ref, logits_ref, i0_ref, i1_ref,
                 w0_ref, w1_ref, c0_ref, call_ref, psum_ref, bl_ref):
    i = pl.program_id(0)
    nsteps = pl.num_programs(0)
    tb = x_ref.shape[0]

    lp = jnp.dot(x_ref[...], wgt_ref[...], preferred_element_type=jnp.float32)
    lanes = jax.lax.broadcasted_iota(jnp.int32, lp.shape, 1)
    valid = lanes < E
    l = jnp.where(valid, lp, NEG)

    m0 = jnp.max(l, axis=1, keepdims=True)
    i0 = jnp.min(jnp.where(l == m0, lanes, 127), axis=1, keepdims=True)
    l2 = jnp.where(lanes == i0, NEG, l)
    m1 = jnp.max(l2, axis=1, keepdims=True)
    i1 = jnp.min(jnp.where(l2 == m1, lanes, 127), axis=1, keepdims=True)

    w0 = jax.nn.sigmoid(m0 - m1)

    oh0 = (lanes == i0).astype(jnp.float32)
    oh1 = (lanes == i1).astype(jnp.float32)

    logits_ref[...] = lp[:, :E]
    i0_ref[...] = i0
    i1_ref[...] = i1
    w0_ref[...] = w0
    w1_ref[...] = 1.0 - w0

    p = jnp.where(valid, jnp.exp(l - m0), 0.0)
    p = p / jnp.sum(p, axis=1, keepdims=True)

    c0_part = jnp.sum(oh0, axis=0, keepdims=True)
    call_part = c0_part + jnp.sum(oh1, axis=0, keepdims=True)
    psum_part = jnp.sum(p, axis=0, keepdims=True)

    @pl.when(i == 0)
    def _init():
        c0_ref[...] = c0_part
        call_ref[...] = call_part
        psum_ref[...] = psum_part

    @pl.when(i > 0)
    def _acc():
        c0_ref[...] += c0_part
        call_ref[...] += call_part
        psum_ref[...] += psum_part

    @pl.when(i == nsteps - 1)
    def _fin():
        t_total = jnp.float32(nsteps * tb)
        bl = (jnp.float32(E) / (t_total * t_total)) * jnp.sum(
            call_ref[...] * psum_ref[...])
        bl_ref[...] = jnp.reshape(bl, (1, 1))


def _run_router(x, Wg):
    t, d = x.shape
    tb = 512 if t % 512 == 0 else t
    wgt = jnp.zeros((d, 128), jnp.float32).at[:, :E].set(Wg.T.astype(jnp.float32))
    return pl.pallas_call(
        _router_body,
        grid=(t // tb,),
        in_specs=[
            pl.BlockSpec((tb, d), lambda i: (i, 0)),
            pl.BlockSpec((d, 128), lambda i: (0, 0)),
        ],
        out_specs=[
            pl.BlockSpec((tb, E), lambda i: (i, 0)),
            pl.BlockSpec((tb, 1), lambda i: (i, 0)),
            pl.BlockSpec((tb, 1), lambda i: (i, 0)),
            pl.BlockSpec((tb, 1), lambda i: (i, 0)),
            pl.BlockSpec((tb, 1), lambda i: (i, 0)),
            pl.BlockSpec((1, 128), lambda i: (0, 0)),
            pl.BlockSpec((1, 128), lambda i: (0, 0)),
            pl.BlockSpec((1, 128), lambda i: (0, 0)),
            pl.BlockSpec((1, 1), lambda i: (0, 0)),
        ],
        out_shape=[
            jax.ShapeDtypeStruct((t, E), jnp.float32),
            jax.ShapeDtypeStruct((t, 1), jnp.int32),
            jax.ShapeDtypeStruct((t, 1), jnp.int32),
            jax.ShapeDtypeStruct((t, 1), jnp.float32),
            jax.ShapeDtypeStruct((t, 1), jnp.float32),
            jax.ShapeDtypeStruct((1, 128), jnp.float32),
            jax.ShapeDtypeStruct((1, 128), jnp.float32),
            jax.ShapeDtypeStruct((1, 128), jnp.float32),
            jax.ShapeDtypeStruct((1, 1), jnp.float32),
        ],
    )(x, wgt)


def _rank_body(i0_ref, i1_ref, off0_ref, off1_ref, p0_ref, p1_ref,
               acc0_ref, acc1_ref, tri_ref):
    i = pl.program_id(0)
    tb = i0_ref.shape[0]

    @pl.when(i == 0)
    def _init():
        r = jax.lax.broadcasted_iota(jnp.int32, (tb, tb), 0)
        c = jax.lax.broadcasted_iota(jnp.int32, (tb, tb), 1)
        tri_ref[...] = (c <= r).astype(jnp.float32)
        acc0_ref[...] = jnp.zeros_like(acc0_ref)
        acc1_ref[...] = jnp.zeros_like(acc1_ref)

    lanes = jax.lax.broadcasted_iota(jnp.int32, (tb, 128), 1)
    oh0 = (lanes == i0_ref[...]).astype(jnp.float32)
    oh1 = (lanes == i1_ref[...]).astype(jnp.float32)
    cum0 = jnp.dot(tri_ref[...], oh0, preferred_element_type=jnp.float32)
    cum1 = jnp.dot(tri_ref[...], oh1, preferred_element_type=jnp.float32)

    pos0 = jnp.sum(oh0 * (off0_ref[...] + acc0_ref[...] + cum0 - 1.0),
                   axis=1, keepdims=True)
    pos1 = jnp.sum(oh1 * (off1_ref[...] + acc1_ref[...] + cum1 - 1.0),
                   axis=1, keepdims=True)
    p0_ref[...] = pos0.astype(jnp.int32)
    p1_ref[...] = pos1.astype(jnp.int32)

    acc0_ref[...] += jnp.sum(oh0, axis=0, keepdims=True)
    acc1_ref[...] += jnp.sum(oh1, axis=0, keepdims=True)


def _run_rank(i0, i1, off0, off1):
    t = i0.shape[0]
    tb = 1024 if t % 1024 == 0 else t
    return pl.pallas_call(
        _rank_body,
        grid=(t // tb,),
        in_specs=[
            pl.BlockSpec((tb, 1), lambda i: (i, 0)),
            pl.BlockSpec((tb, 1), lambda i: (i, 0)),
            pl.BlockSpec((1, 128), lambda i: (0, 0)),
            pl.BlockSpec((1, 128), lambda i: (0, 0)),
        ],
        out_specs=[
            pl.BlockSpec((tb, 1), lambda i: (i, 0)),
            pl.BlockSpec((tb, 1), lambda i: (i, 0)),
        ],
        out_shape=[
            jax.ShapeDtypeStruct((t, 1), jnp.int32),
            jax.ShapeDtypeStruct((t, 1), jnp.int32),
        ],
        scratch_shapes=[
            pltpu.VMEM((1, 128), jnp.float32),
            pltpu.VMEM((1, 128), jnp.float32),
            pltpu.VMEM((tb, tb), jnp.float32),
        ],
    )(i0, i1, off0, off1)


def _sc_mesh():
    return plsc.VectorSubcoreMesh(core_axis_name="c", subcore_axis_name="s")


def _run_dispatch(x, p0_2d, p1_2d, nslot, chunk_rows=16):
    t, d = x.shape
    info = plsc.get_sparse_core_info()
    nw = info.num_cores * info.num_subcores
    per_w = t // nw
    nch = per_w // chunk_rows

    @functools.partial(
        pl.kernel,
        out_type=jax.ShapeDtypeStruct((nslot, d), x.dtype),
        mesh=_sc_mesh(),
        scratch_types=[
            pltpu.VMEM((nch, chunk_rows), jnp.int32),
            pltpu.VMEM((nch, chunk_rows), jnp.int32),
            pltpu.VMEM((chunk_rows, d), x.dtype),
            pltpu.VMEM((chunk_rows, d), x.dtype),
            pltpu.SemaphoreType.DMA,
            pltpu.SemaphoreType.DMA,
        ],
    )
    def run(x_h, p0_h, p1_h, xs_h, idx0_v, idx1_v, buf0, buf1, sem0, sem1):
        wid = lax.axis_index("s") * info.num_cores + lax.axis_index("c")
        base = wid * per_w
        pltpu.sync_copy(p0_h.at[pl.ds(wid * nch, nch)], idx0_v)
        pltpu.sync_copy(p1_h.at[pl.ds(wid * nch, nch)], idx1_v)
        bufs = (buf0, buf1)
        sems = (sem0, sem1)
        cps = []
        for c in range(nch):
            b = bufs[c % 2]
            if c >= 2:
                cps[2 * (c - 2)].wait()
                cps[2 * (c - 2) + 1].wait()
            pltpu.sync_copy(x_h.at[pl.ds(base + c * chunk_rows, chunk_rows)], b)
            cps.append(pltpu.async_copy(b, xs_h.at[idx0_v.at[c]], sems[c % 2]))
            cps.append(pltpu.async_copy(b, xs_h.at[idx1_v.at[c]], sems[c % 2]))
        for c in range(max(nch - 2, 0), nch):
            cps[2 * c].wait()
            cps[2 * c + 1].wait()

    return run(x, p0_2d, p1_2d)


def _run_gather2(table, idx0, idx1, chunk_rows=8):
    n = idx0.shape[0]
    d = table.shape[1]
    info = plsc.get_sparse_core_info()
    nw = info.num_cores * info.num_subcores
    per_w = n // nw
    nch = per_w // chunk_rows

    @functools.partial(
        pl.kernel,
        out_type=[
            jax.ShapeDtypeStruct((n, d), table.dtype),
            jax.ShapeDtypeStruct((n, d), table.dtype),
        ],
        mesh=_sc_mesh(),
        scratch_types=[
            pltpu.VMEM((per_w,), jnp.int32),
            pltpu.VMEM((per_w,), jnp.int32),
            pltpu.VMEM((chunk_rows, d), table.dtype),
            pltpu.VMEM((chunk_rows, d), table.dtype),
            pltpu.VMEM((chunk_rows, d), table.dtype),
            pltpu.VMEM((chunk_rows, d), table.dtype),
            pltpu.SemaphoreType.DMA,
            pltpu.SemaphoreType.DMA,
        ],
    )
    def run(tab_h, idx0_h, idx1_h, ya_h, yb_h, idx0_v, idx1_v,
            bufa0, bufa1, bufb0, bufb1, sema, semb):
        wid = lax.axis_index("s") * info.num_cores + lax.axis_index("c")
        base = wid * per_w
        pltpu.sync_copy(idx0_h.at[pl.ds(base, per_w)], idx0_v)
        pltpu.sync_copy(idx1_h.at[pl.ds(base, per_w)], idx1_v)
        bas = (bufa0, bufa1)
        bbs = (bufb0, bufb1)
        cpa, cpb = [], []
        for c in range(nch):
            sl = pl.ds(c * chunk_rows, chunk_rows)
            cpa.append(pltpu.async_copy(tab_h.at[idx0_v.at[sl]],
                                        bas[c % 2], sema))
            cpb.append(pltpu.async_copy(tab_h.at[idx1_v.at[sl]],
                                        bbs[c % 2], semb))
            if c >= 1:
                osl = pl.ds(base + (c - 1) * chunk_rows, chunk_rows)
                cpa[c - 1].wait()
                pltpu.sync_copy(bas[(c - 1) % 2], ya_h.at[osl])
                cpb[c - 1].wait()
                pltpu.sync_copy(bbs[(c - 1) % 2], yb_h.at[osl])
        osl = pl.ds(base + (nch - 1) * chunk_rows, chunk_rows)
        cpa[nch - 1].wait()
        pltpu.sync_copy(bas[(nch - 1) % 2], ya_h.at[osl])
        cpb[nch - 1].wait()
        pltpu.sync_copy(bbs[(nch - 1) % 2], yb_h.at[osl])

    return run(table, idx0, idx1)


def _mlp_body(be_ref, xs_ref, gw_ref, pw_ref, ow_ref, ys_ref):
    xb = xs_ref[...].astype(jnp.bfloat16)
    g = jnp.dot(xb, gw_ref[0], preferred_element_type=jnp.float32)
    p = jnp.dot(xb, pw_ref[0], preferred_element_type=jnp.float32)
    h = g * (p * jax.nn.sigmoid(p))
    ys_ref[...] = jnp.dot(h.astype(jnp.bfloat16), ow_ref[0],
                          preferred_element_type=jnp.float32)


def _run_mlp(xs, gwb, pwb, owb, block_expert):
    nslot, d = xs.shape
    fdim = gwb.shape[2]
    nblk = nslot // BLK
    grid_spec = pltpu.PrefetchScalarGridSpec(
        num_scalar_prefetch=1,
        grid=(nblk,),
        in_specs=[
            pl.BlockSpec((BLK, d), lambda b, be: (b, 0)),
            pl.BlockSpec((1, d, fdim), lambda b, be: (be[b], 0, 0)),
            pl.BlockSpec((1, d, fdim), lambda b, be: (be[b], 0, 0)),
            pl.BlockSpec((1, fdim, d), lambda b, be: (be[b], 0, 0)),
        ],
        out_specs=pl.BlockSpec((BLK, d), lambda b, be: (b, 0)),
    )
    return pl.pallas_call(
        _mlp_body,
        grid_spec=grid_spec,
        out_shape=jax.ShapeDtypeStruct((nslot, d), jnp.float32),
    )(block_expert, xs, gwb, pwb, owb)


def _combine_body(ya_ref, yb_ref, w0_ref, w1_ref, out_ref):
    out_ref[...] = w0_ref[...] * ya_ref[...] + w1_ref[...] * yb_ref[...]


def _run_combine(ya, yb, w0, w1):
    t, d = ya.shape
    tb = 512 if t % 512 == 0 else t
    return pl.pallas_call(
        _combine_body,
        grid=(t // tb,),
        in_specs=[
            pl.BlockSpec((tb, d), lambda i: (i, 0)),
            pl.BlockSpec((tb, d), lambda i: (i, 0)),
            pl.BlockSpec((tb, 1), lambda i: (i, 0)),
            pl.BlockSpec((tb, 1), lambda i: (i, 0)),
        ],
        out_specs=pl.BlockSpec((tb, d), lambda i: (i, 0)),
        out_shape=jax.ShapeDtypeStruct((t, d), jnp.float32),
    )(ya, yb, w0, w1)


def kernel(hidden_states, Wg, gw, pw, ow):
    b, s, d = hidden_states.shape
    x = hidden_states.reshape(-1, d).astype(jnp.float32)
    t = x.shape[0]
    ne, _, fdim = gw.shape
    nslot = TOPK * t + ne * BLK

    (logits, i0, i1, w0, w1, c0, call, _psum, bl) = _run_router(x, Wg)

    c0v = c0[0, :ne].astype(jnp.int32)
    callv = call[0, :ne].astype(jnp.int32)
    padded = ((callv + (BLK - 1)) // BLK) * BLK
    base = jnp.cumsum(padded) - padded
    off0 = jnp.zeros((1, 128), jnp.float32).at[0, :ne].set(base.astype(jnp.float32))
    off1 = jnp.zeros((1, 128), jnp.float32).at[0, :ne].set(
        (base + c0v).astype(jnp.float32))
    ends = (base + padded) // BLK
    nblk = nslot // BLK
    block_expert = jnp.minimum(
        jnp.sum(jnp.arange(nblk)[:, None] >= ends[None, :], axis=1),
        ne - 1).astype(jnp.int32)

    p0, p1 = _run_rank(i0, i1, off0, off1)
    p0f, p1f = p0.reshape(t), p1.reshape(t)

    ch = 16
    xs = _run_dispatch(x, p0f.reshape(t // ch, ch), p1f.reshape(t // ch, ch),
                       nslot, chunk_rows=ch)
    ys = _run_mlp(xs, gw.astype(jnp.bfloat16), pw.astype(jnp.bfloat16),
                  ow.astype(jnp.bfloat16), block_expert)
    ya, yb = _run_gather2(ys, p0f, p1f)
    out = _run_combine(ya, yb, w0, w1)

    return (out.reshape(b, s, d), logits, bl[0, 0])


# speedup vs baseline: 1.9262x; 1.0947x over previous
import functools

import jax
import jax.numpy as jnp
from jax import lax
from jax.experimental import pallas as pl
from jax.experimental.pallas import tpu as pltpu
from jax.experimental.pallas import tpu_sc as plsc

E = 8
TOPK = 2
NEG = -1e30
BLK = 512


def _router_body(x_ref, wgt_ref, logits_ref, i0_ref, i1_ref,
                 w0e_ref, w1e_ref, c0_ref, call_ref, psum_ref, bl_ref):
    i = pl.program_id(0)
    nsteps = pl.num_programs(0)
    tb = x_ref.shape[0]

    lp = jnp.dot(x_ref[...], wgt_ref[...], preferred_element_type=jnp.float32)
    lanes = jax.lax.broadcasted_iota(jnp.int32, lp.shape, 1)
    valid = lanes < E
    l = jnp.where(valid, lp, NEG)

    m0 = jnp.max(l, axis=1, keepdims=True)
    i0 = jnp.min(jnp.where(l == m0, lanes, 127), axis=1, keepdims=True)
    l2 = jnp.where(lanes == i0, NEG, l)
    m1 = jnp.max(l2, axis=1, keepdims=True)
    i1 = jnp.min(jnp.where(l2 == m1, lanes, 127), axis=1, keepdims=True)

    w0 = jax.nn.sigmoid(m0 - m1)

    oh0 = (lanes == i0).astype(jnp.float32)
    oh1 = (lanes == i1).astype(jnp.float32)

    logits_ref[...] = lp[:, :E]
    i0_ref[...] = i0
    i1_ref[...] = i1
    w0e_ref[...] = jnp.broadcast_to(w0, w0e_ref.shape)
    w1e_ref[...] = jnp.broadcast_to(1.0 - w0, w1e_ref.shape)

    p = jnp.where(valid, jnp.exp(l - m0), 0.0)
    p = p / jnp.sum(p, axis=1, keepdims=True)

    c0_part = jnp.sum(oh0, axis=0, keepdims=True)
    call_part = c0_part + jnp.sum(oh1, axis=0, keepdims=True)
    psum_part = jnp.sum(p, axis=0, keepdims=True)

    @pl.when(i == 0)
    def _init():
        c0_ref[...] = c0_part
        call_ref[...] = call_part
        psum_ref[...] = psum_part

    @pl.when(i > 0)
    def _acc():
        c0_ref[...] += c0_part
        call_ref[...] += call_part
        psum_ref[...] += psum_part

    @pl.when(i == nsteps - 1)
    def _fin():
        t_total = jnp.float32(nsteps * tb)
        bl = (jnp.float32(E) / (t_total * t_total)) * jnp.sum(
            call_ref[...] * psum_ref[...])
        bl_ref[...] = jnp.reshape(bl, (1, 1))


def _run_router(x, Wg):
    t, d = x.shape
    tb = 512 if t % 512 == 0 else t
    wgt = jnp.zeros((d, 128), jnp.float32).at[:, :E].set(Wg.T.astype(jnp.float32))
    return pl.pallas_call(
        _router_body,
        grid=(t // tb,),
        in_specs=[
            pl.BlockSpec((tb, d), lambda i: (i, 0)),
            pl.BlockSpec((d, 128), lambda i: (0, 0)),
        ],
        out_specs=[
            pl.BlockSpec((tb, E), lambda i: (i, 0)),
            pl.BlockSpec((tb, 1), lambda i: (i, 0)),
            pl.BlockSpec((tb, 1), lambda i: (i, 0)),
            pl.BlockSpec((tb, 16), lambda i: (i, 0)),
            pl.BlockSpec((tb, 16), lambda i: (i, 0)),
            pl.BlockSpec((1, 128), lambda i: (0, 0)),
            pl.BlockSpec((1, 128), lambda i: (0, 0)),
            pl.BlockSpec((1, 128), lambda i: (0, 0)),
            pl.BlockSpec((1, 1), lambda i: (0, 0)),
        ],
        out_shape=[
            jax.ShapeDtypeStruct((t, E), jnp.float32),
            jax.ShapeDtypeStruct((t, 1), jnp.int32),
            jax.ShapeDtypeStruct((t, 1), jnp.int32),
            jax.ShapeDtypeStruct((t, 16), jnp.float32),
            jax.ShapeDtypeStruct((t, 16), jnp.float32),
            jax.ShapeDtypeStruct((1, 128), jnp.float32),
            jax.ShapeDtypeStruct((1, 128), jnp.float32),
            jax.ShapeDtypeStruct((1, 128), jnp.float32),
            jax.ShapeDtypeStruct((1, 1), jnp.float32),
        ],
    )(x, wgt)


def _rank_body(i0_ref, i1_ref, off0_ref, off1_ref, p0_ref, p1_ref,
               acc0_ref, acc1_ref, tri_ref):
    i = pl.program_id(0)
    tb = i0_ref.shape[0]

    @pl.when(i == 0)
    def _init():
        r = jax.lax.broadcasted_iota(jnp.int32, (tb, tb), 0)
        c = jax.lax.broadcasted_iota(jnp.int32, (tb, tb), 1)
        tri_ref[...] = (c <= r).astype(jnp.float32)
        acc0_ref[...] = jnp.zeros_like(acc0_ref)
        acc1_ref[...] = jnp.zeros_like(acc1_ref)

    lanes = jax.lax.broadcasted_iota(jnp.int32, (tb, 128), 1)
    oh0 = (lanes == i0_ref[...]).astype(jnp.float32)
    oh1 = (lanes == i1_ref[...]).astype(jnp.float32)
    cum0 = jnp.dot(tri_ref[...], oh0, preferred_element_type=jnp.float32)
    cum1 = jnp.dot(tri_ref[...], oh1, preferred_element_type=jnp.float32)

    pos0 = jnp.sum(oh0 * (off0_ref[...] + acc0_ref[...] + cum0 - 1.0),
                   axis=1, keepdims=True)
    pos1 = jnp.sum(oh1 * (off1_ref[...] + acc1_ref[...] + cum1 - 1.0),
                   axis=1, keepdims=True)
    p0_ref[...] = pos0.astype(jnp.int32)
    p1_ref[...] = pos1.astype(jnp.int32)

    acc0_ref[...] += jnp.sum(oh0, axis=0, keepdims=True)
    acc1_ref[...] += jnp.sum(oh1, axis=0, keepdims=True)


def _run_rank(i0, i1, off0, off1):
    t = i0.shape[0]
    tb = 1024 if t % 1024 == 0 else t
    return pl.pallas_call(
        _rank_body,
        grid=(t // tb,),
        in_specs=[
            pl.BlockSpec((tb, 1), lambda i: (i, 0)),
            pl.BlockSpec((tb, 1), lambda i: (i, 0)),
            pl.BlockSpec((1, 128), lambda i: (0, 0)),
            pl.BlockSpec((1, 128), lambda i: (0, 0)),
        ],
        out_specs=[
            pl.BlockSpec((tb, 1), lambda i: (i, 0)),
            pl.BlockSpec((tb, 1), lambda i: (i, 0)),
        ],
        out_shape=[
            jax.ShapeDtypeStruct((t, 1), jnp.int32),
            jax.ShapeDtypeStruct((t, 1), jnp.int32),
        ],
        scratch_shapes=[
            pltpu.VMEM((1, 128), jnp.float32),
            pltpu.VMEM((1, 128), jnp.float32),
            pltpu.VMEM((tb, tb), jnp.float32),
        ],
    )(i0, i1, off0, off1)


def _sc_mesh():
    return plsc.VectorSubcoreMesh(core_axis_name="c", subcore_axis_name="s")


def _run_dispatch(x, p0_2d, p1_2d, nslot, chunk_rows=16):
    t, d = x.shape
    info = plsc.get_sparse_core_info()
    nw = info.num_cores * info.num_subcores
    per_w = t // nw
    nch = per_w // chunk_rows

    @functools.partial(
        pl.kernel,
        out_type=jax.ShapeDtypeStruct((nslot, d), x.dtype),
        mesh=_sc_mesh(),
        scratch_types=[
            pltpu.VMEM((nch, chunk_rows), jnp.int32),
            pltpu.VMEM((nch, chunk_rows), jnp.int32),
            pltpu.VMEM((chunk_rows, d), x.dtype),
            pltpu.VMEM((chunk_rows, d), x.dtype),
            pltpu.SemaphoreType.DMA,
            pltpu.SemaphoreType.DMA,
        ],
    )
    def run(x_h, p0_h, p1_h, xs_h, idx0_v, idx1_v, buf0, buf1, sem0, sem1):
        wid = lax.axis_index("s") * info.num_cores + lax.axis_index("c")
        base = wid * per_w
        pltpu.sync_copy(p0_h.at[pl.ds(wid * nch, nch)], idx0_v)
        pltpu.sync_copy(p1_h.at[pl.ds(wid * nch, nch)], idx1_v)
        bufs = (buf0, buf1)
        sems = (sem0, sem1)
        cps = []
        for c in range(nch):
            b = bufs[c % 2]
            if c >= 2:
                cps[2 * (c - 2)].wait()
                cps[2 * (c - 2) + 1].wait()
            pltpu.sync_copy(x_h.at[pl.ds(base + c * chunk_rows, chunk_rows)], b)
            cps.append(pltpu.async_copy(b, xs_h.at[idx0_v.at[c]], sems[c % 2]))
            cps.append(pltpu.async_copy(b, xs_h.at[idx1_v.at[c]], sems[c % 2]))
        for c in range(max(nch - 2, 0), nch):
            cps[2 * c].wait()
            cps[2 * c + 1].wait()

    return run(x, p0_2d, p1_2d)


def _run_gather_combine(table, idx0, idx1, w0e, chunk_rows=8):
    n = idx0.shape[0]
    d = table.shape[1]
    info = plsc.get_sparse_core_info()
    nw = info.num_cores * info.num_subcores
    per_w = n // nw
    nch = per_w // chunk_rows
    nseg = d // 16

    @functools.partial(
        pl.kernel,
        out_type=jax.ShapeDtypeStruct((n, d), table.dtype),
        mesh=_sc_mesh(),
        scratch_types=[
            pltpu.VMEM((per_w,), jnp.int32),
            pltpu.VMEM((per_w,), jnp.int32),
            pltpu.VMEM((per_w, 16), jnp.float32),
            pltpu.VMEM((chunk_rows, d), table.dtype),
            pltpu.VMEM((chunk_rows, d), table.dtype),
            pltpu.VMEM((chunk_rows, d), table.dtype),
            pltpu.VMEM((chunk_rows, d), table.dtype),
            pltpu.SemaphoreType.DMA,
            pltpu.SemaphoreType.DMA,
            pltpu.SemaphoreType.DMA,
        ],
    )
    def run(tab_h, idx0_h, idx1_h, w0_h, out_h, idx0_v, idx1_v,
            w0_v, bufa0, bufa1, bufb0, bufb1, sema, semb, semo):
        wid = lax.axis_index("s") * info.num_cores + lax.axis_index("c")
        base = wid * per_w
        pltpu.sync_copy(idx0_h.at[pl.ds(base, per_w)], idx0_v)
        pltpu.sync_copy(idx1_h.at[pl.ds(base, per_w)], idx1_v)
        pltpu.sync_copy(w0_h.at[pl.ds(base, per_w)], w0_v)
        bas = (bufa0, bufa1)
        bbs = (bufb0, bufb1)

        def fma(c):
            par = c % 2
            ba, bb = bas[par], bbs[par]
            w0r = [w0_v[c * chunk_rows + j] for j in range(chunk_rows)]
            w1r = [1.0 - w for w in w0r]

            def seg(i, carry):
                sl = pl.ds(i * 16, 16)
                for j in range(chunk_rows):
                    ba[j, sl] = w0r[j] * ba[j, sl] + w1r[j] * bb[j, sl]
                return carry

            lax.fori_loop(0, nseg, seg, 0)

        cpa, cpb, cpo = [], [], []
        for c in range(nch):
            if c >= 2:
                cpo[c - 2].wait()
            sl = pl.ds(c * chunk_rows, chunk_rows)
            cpa.append(pltpu.async_copy(tab_h.at[idx0_v.at[sl]],
                                        bas[c % 2], sema))
            cpb.append(pltpu.async_copy(tab_h.at[idx1_v.at[sl]],
                                        bbs[c % 2], semb))
            if c >= 1:
                cpa[c - 1].wait()
                cpb[c - 1].wait()
                fma(c - 1)
                cpo.append(pltpu.async_copy(
                    bas[(c - 1) % 2],
                    out_h.at[pl.ds(base + (c - 1) * chunk_rows, chunk_rows)],
                    semo))
        cpa[nch - 1].wait()
        cpb[nch - 1].wait()
        fma(nch - 1)
        cpo.append(pltpu.async_copy(
            bas[(nch - 1) % 2],
            out_h.at[pl.ds(base + (nch - 1) * chunk_rows, chunk_rows)],
            semo))
        cpo[nch - 2].wait()
        cpo[nch - 1].wait()

    return run(table, idx0, idx1, w0e)


def _mlp_body(be_ref, us_ref, xs_ref, gw_ref, pw_ref, ow_ref, ys_ref):
    b = pl.program_id(0)

    @pl.when(b < us_ref[0])
    def _compute():
        xb = xs_ref[...].astype(jnp.bfloat16)
        g = jnp.dot(xb, gw_ref[0], preferred_element_type=jnp.float32)
        p = jnp.dot(xb, pw_ref[0], preferred_element_type=jnp.float32)
        h = g * (p * jax.nn.sigmoid(p))
        ys_ref[...] = jnp.dot(h.astype(jnp.bfloat16), ow_ref[0],
                              preferred_element_type=jnp.float32)


def _run_mlp(xs, gwb, pwb, owb, block_expert, used):
    nslot, d = xs.shape
    fdim = gwb.shape[2]
    nblk = nslot // BLK
    grid_spec = pltpu.PrefetchScalarGridSpec(
        num_scalar_prefetch=2,
        grid=(nblk,),
        in_specs=[
            pl.BlockSpec((BLK, d), lambda b, be, us: (b, 0)),
            pl.BlockSpec((1, d, fdim), lambda b, be, us: (be[b], 0, 0)),
            pl.BlockSpec((1, d, fdim), lambda b, be, us: (be[b], 0, 0)),
            pl.BlockSpec((1, fdim, d), lambda b, be, us: (be[b], 0, 0)),
        ],
        out_specs=pl.BlockSpec((BLK, d), lambda b, be, us: (b, 0)),
    )
    return pl.pallas_call(
        _mlp_body,
        grid_spec=grid_spec,
        out_shape=jax.ShapeDtypeStruct((nslot, d), jnp.float32),
    )(block_expert, used, xs, gwb, pwb, owb)


def kernel(hidden_states, Wg, gw, pw, ow):
    b, s, d = hidden_states.shape
    x = hidden_states.reshape(-1, d).astype(jnp.float32)
    t = x.shape[0]
    ne, _, fdim = gw.shape
    nslot = TOPK * t + ne * BLK

    (logits, i0, i1, w0e, w1e, c0, call, _psum, bl) = _run_router(x, Wg)

    c0v = c0[0, :ne].astype(jnp.int32)
    callv = call[0, :ne].astype(jnp.int32)
    padded = ((callv + (BLK - 1)) // BLK) * BLK
    base = jnp.cumsum(padded) - padded
    off0 = jnp.zeros((1, 128), jnp.float32).at[0, :ne].set(base.astype(jnp.float32))
    off1 = jnp.zeros((1, 128), jnp.float32).at[0, :ne].set(
        (base + c0v).astype(jnp.float32))
    ends = (base + padded) // BLK
    nblk = nslot // BLK
    block_expert = jnp.minimum(
        jnp.sum(jnp.arange(nblk)[:, None] >= ends[None, :], axis=1),
        ne - 1).astype(jnp.int32)
    used = (jnp.sum(padded) // BLK).astype(jnp.int32).reshape(1)

    p0, p1 = _run_rank(i0, i1, off0, off1)
    p0f, p1f = p0.reshape(t), p1.reshape(t)

    ch = 16
    xs = _run_dispatch(x, p0f.reshape(t // ch, ch), p1f.reshape(t // ch, ch),
                       nslot, chunk_rows=ch)
    ys = _run_mlp(xs, gw.astype(jnp.bfloat16), pw.astype(jnp.bfloat16),
                  ow.astype(jnp.bfloat16), block_expert, used)
    out = _run_gather_combine(ys, p0f, p1f, w0e)

    return (out.reshape(b, s, d), logits, bl[0, 0])


# speedup vs baseline: 1.9546x; 1.0147x over previous
import functools

import jax
import jax.numpy as jnp
from jax import lax
from jax.experimental import pallas as pl
from jax.experimental.pallas import tpu as pltpu
from jax.experimental.pallas import tpu_sc as plsc

E = 8
TOPK = 2
NEG = -1e30
BLK = 512


def _router_body(x_ref, wgt_ref, logits_ref, i0_ref, i1_ref,
                 w0e_ref, w1e_ref, c0_ref, call_ref, psum_ref, bl_ref):
    i = pl.program_id(0)
    nsteps = pl.num_programs(0)
    tb = x_ref.shape[0]

    lp = jnp.dot(x_ref[...], wgt_ref[...], preferred_element_type=jnp.float32)
    lanes = jax.lax.broadcasted_iota(jnp.int32, lp.shape, 1)
    valid = lanes < E
    l = jnp.where(valid, lp, NEG)

    m0 = jnp.max(l, axis=1, keepdims=True)
    i0 = jnp.min(jnp.where(l == m0, lanes, 127), axis=1, keepdims=True)
    l2 = jnp.where(lanes == i0, NEG, l)
    m1 = jnp.max(l2, axis=1, keepdims=True)
    i1 = jnp.min(jnp.where(l2 == m1, lanes, 127), axis=1, keepdims=True)

    w0 = jax.nn.sigmoid(m0 - m1)

    oh0 = (lanes == i0).astype(jnp.float32)
    oh1 = (lanes == i1).astype(jnp.float32)

    logits_ref[...] = lp[:, :E]
    i0_ref[...] = i0
    i1_ref[...] = i1
    w0e_ref[...] = jnp.broadcast_to(w0, w0e_ref.shape)
    w1e_ref[...] = jnp.broadcast_to(1.0 - w0, w1e_ref.shape)

    p = jnp.where(valid, jnp.exp(l - m0), 0.0)
    p = p / jnp.sum(p, axis=1, keepdims=True)

    c0_part = jnp.sum(oh0, axis=0, keepdims=True)
    call_part = c0_part + jnp.sum(oh1, axis=0, keepdims=True)
    psum_part = jnp.sum(p, axis=0, keepdims=True)

    @pl.when(i == 0)
    def _init():
        c0_ref[...] = c0_part
        call_ref[...] = call_part
        psum_ref[...] = psum_part

    @pl.when(i > 0)
    def _acc():
        c0_ref[...] += c0_part
        call_ref[...] += call_part
        psum_ref[...] += psum_part

    @pl.when(i == nsteps - 1)
    def _fin():
        t_total = jnp.float32(nsteps * tb)
        bl = (jnp.float32(E) / (t_total * t_total)) * jnp.sum(
            call_ref[...] * psum_ref[...])
        bl_ref[...] = jnp.reshape(bl, (1, 1))


def _run_router(x, Wg):
    t, d = x.shape
    tb = 512 if t % 512 == 0 else t
    wgt = jnp.zeros((d, 128), jnp.float32).at[:, :E].set(Wg.T.astype(jnp.float32))
    return pl.pallas_call(
        _router_body,
        grid=(t // tb,),
        in_specs=[
            pl.BlockSpec((tb, d), lambda i: (i, 0)),
            pl.BlockSpec((d, 128), lambda i: (0, 0)),
        ],
        out_specs=[
            pl.BlockSpec((tb, E), lambda i: (i, 0)),
            pl.BlockSpec((tb, 1), lambda i: (i, 0)),
            pl.BlockSpec((tb, 1), lambda i: (i, 0)),
            pl.BlockSpec((tb, 16), lambda i: (i, 0)),
            pl.BlockSpec((tb, 16), lambda i: (i, 0)),
            pl.BlockSpec((1, 128), lambda i: (0, 0)),
            pl.BlockSpec((1, 128), lambda i: (0, 0)),
            pl.BlockSpec((1, 128), lambda i: (0, 0)),
            pl.BlockSpec((1, 1), lambda i: (0, 0)),
        ],
        out_shape=[
            jax.ShapeDtypeStruct((t, E), jnp.float32),
            jax.ShapeDtypeStruct((t, 1), jnp.int32),
            jax.ShapeDtypeStruct((t, 1), jnp.int32),
            jax.ShapeDtypeStruct((t, 16), jnp.float32),
            jax.ShapeDtypeStruct((t, 16), jnp.float32),
            jax.ShapeDtypeStruct((1, 128), jnp.float32),
            jax.ShapeDtypeStruct((1, 128), jnp.float32),
            jax.ShapeDtypeStruct((1, 128), jnp.float32),
            jax.ShapeDtypeStruct((1, 1), jnp.float32),
        ],
    )(x, wgt)


def _rank_body(i0_ref, i1_ref, off0_ref, off1_ref, p0_ref, p1_ref,
               acc0_ref, acc1_ref, tri_ref):
    i = pl.program_id(0)
    tb = i0_ref.shape[0]

    @pl.when(i == 0)
    def _init():
        r = jax.lax.broadcasted_iota(jnp.int32, (tb, tb), 0)
        c = jax.lax.broadcasted_iota(jnp.int32, (tb, tb), 1)
        tri_ref[...] = (c <= r).astype(jnp.float32)
        acc0_ref[...] = jnp.zeros_like(acc0_ref)
        acc1_ref[...] = jnp.zeros_like(acc1_ref)

    lanes = jax.lax.broadcasted_iota(jnp.int32, (tb, 128), 1)
    oh0 = (lanes == i0_ref[...]).astype(jnp.float32)
    oh1 = (lanes == i1_ref[...]).astype(jnp.float32)
    cum0 = jnp.dot(tri_ref[...], oh0, preferred_element_type=jnp.float32)
    cum1 = jnp.dot(tri_ref[...], oh1, preferred_element_type=jnp.float32)

    pos0 = jnp.sum(oh0 * (off0_ref[...] + acc0_ref[...] + cum0 - 1.0),
                   axis=1, keepdims=True)
    pos1 = jnp.sum(oh1 * (off1_ref[...] + acc1_ref[...] + cum1 - 1.0),
                   axis=1, keepdims=True)
    p0_ref[...] = pos0.astype(jnp.int32)
    p1_ref[...] = pos1.astype(jnp.int32)

    acc0_ref[...] += jnp.sum(oh0, axis=0, keepdims=True)
    acc1_ref[...] += jnp.sum(oh1, axis=0, keepdims=True)


def _run_rank(i0, i1, off0, off1):
    t = i0.shape[0]
    tb = 1024 if t % 1024 == 0 else t
    return pl.pallas_call(
        _rank_body,
        grid=(t // tb,),
        in_specs=[
            pl.BlockSpec((tb, 1), lambda i: (i, 0)),
            pl.BlockSpec((tb, 1), lambda i: (i, 0)),
            pl.BlockSpec((1, 128), lambda i: (0, 0)),
            pl.BlockSpec((1, 128), lambda i: (0, 0)),
        ],
        out_specs=[
            pl.BlockSpec((tb, 1), lambda i: (i, 0)),
            pl.BlockSpec((tb, 1), lambda i: (i, 0)),
        ],
        out_shape=[
            jax.ShapeDtypeStruct((t, 1), jnp.int32),
            jax.ShapeDtypeStruct((t, 1), jnp.int32),
        ],
        scratch_shapes=[
            pltpu.VMEM((1, 128), jnp.float32),
            pltpu.VMEM((1, 128), jnp.float32),
            pltpu.VMEM((tb, tb), jnp.float32),
        ],
    )(i0, i1, off0, off1)


def _sc_mesh():
    return plsc.VectorSubcoreMesh(core_axis_name="c", subcore_axis_name="s")


def _run_dispatch(x, p0_2d, p1_2d, nslot, chunk_rows=16):
    t, d = x.shape
    info = plsc.get_sparse_core_info()
    nw = info.num_cores * info.num_subcores
    per_w = t // nw
    nch = per_w // chunk_rows

    @functools.partial(
        pl.kernel,
        out_type=jax.ShapeDtypeStruct((nslot, d), x.dtype),
        mesh=_sc_mesh(),
        scratch_types=[
            pltpu.VMEM((nch, chunk_rows), jnp.int32),
            pltpu.VMEM((nch, chunk_rows), jnp.int32),
            pltpu.VMEM((chunk_rows, d), x.dtype),
            pltpu.VMEM((chunk_rows, d), x.dtype),
            pltpu.SemaphoreType.DMA,
            pltpu.SemaphoreType.DMA,
        ],
    )
    def run(x_h, p0_h, p1_h, xs_h, idx0_v, idx1_v, buf0, buf1, sem0, sem1):
        wid = lax.axis_index("s") * info.num_cores + lax.axis_index("c")
        base = wid * per_w
        pltpu.sync_copy(p0_h.at[pl.ds(wid * nch, nch)], idx0_v)
        pltpu.sync_copy(p1_h.at[pl.ds(wid * nch, nch)], idx1_v)
        bufs = (buf0, buf1)
        sems = (sem0, sem1)
        cps = []
        for c in range(nch):
            b = bufs[c % 2]
            if c >= 2:
                cps[2 * (c - 2)].wait()
                cps[2 * (c - 2) + 1].wait()
            pltpu.sync_copy(x_h.at[pl.ds(base + c * chunk_rows, chunk_rows)], b)
            cps.append(pltpu.async_copy(b, xs_h.at[idx0_v.at[c]], sems[c % 2]))
            cps.append(pltpu.async_copy(b, xs_h.at[idx1_v.at[c]], sems[c % 2]))
        for c in range(max(nch - 2, 0), nch):
            cps[2 * c].wait()
            cps[2 * c + 1].wait()

    return run(x, p0_2d, p1_2d)


def _run_gather_combine(table, idx0, idx1, w0e, chunk_rows=8):
    n = idx0.shape[0]
    d = table.shape[1]
    info = plsc.get_sparse_core_info()
    nw = info.num_cores * info.num_subcores
    per_w = n // nw
    nch = per_w // chunk_rows
    nseg = d // 16

    @functools.partial(
        pl.kernel,
        out_type=jax.ShapeDtypeStruct((n, d), table.dtype),
        mesh=_sc_mesh(),
        scratch_types=[
            pltpu.VMEM((per_w,), jnp.int32),
            pltpu.VMEM((per_w,), jnp.int32),
            pltpu.VMEM((per_w, 16), jnp.float32),
            pltpu.VMEM((chunk_rows, d), table.dtype),
            pltpu.VMEM((chunk_rows, d), table.dtype),
            pltpu.VMEM((chunk_rows, d), table.dtype),
            pltpu.VMEM((chunk_rows, d), table.dtype),
            pltpu.SemaphoreType.DMA,
            pltpu.SemaphoreType.DMA,
            pltpu.SemaphoreType.DMA,
        ],
    )
    def run(tab_h, idx0_h, idx1_h, w0_h, out_h, idx0_v, idx1_v,
            w0_v, bufa0, bufa1, bufb0, bufb1, sema, semb, semo):
        wid = lax.axis_index("s") * info.num_cores + lax.axis_index("c")
        base = wid * per_w
        pltpu.sync_copy(idx0_h.at[pl.ds(base, per_w)], idx0_v)
        pltpu.sync_copy(idx1_h.at[pl.ds(base, per_w)], idx1_v)
        pltpu.sync_copy(w0_h.at[pl.ds(base, per_w)], w0_v)
        bas = (bufa0, bufa1)
        bbs = (bufb0, bufb1)

        def fma(c):
            par = c % 2
            ba, bb = bas[par], bbs[par]
            w0r = [w0_v[c * chunk_rows + j] for j in range(chunk_rows)]
            w1r = [1.0 - w for w in w0r]

            def seg(i, carry):
                sl = pl.ds(i * 16, 16)
                for j in range(chunk_rows):
                    ba[j, sl] = w0r[j] * ba[j, sl] + w1r[j] * bb[j, sl]
                return carry

            lax.fori_loop(0, nseg, seg, 0)

        cpa, cpb, cpo = [], [], []
        for c in range(nch):
            if c >= 2:
                cpo[c - 2].wait()
            sl = pl.ds(c * chunk_rows, chunk_rows)
            cpa.append(pltpu.async_copy(tab_h.at[idx0_v.at[sl]],
                                        bas[c % 2], sema))
            cpb.append(pltpu.async_copy(tab_h.at[idx1_v.at[sl]],
                                        bbs[c % 2], semb))
            if c >= 1:
                cpa[c - 1].wait()
                cpb[c - 1].wait()
                fma(c - 1)
                cpo.append(pltpu.async_copy(
                    bas[(c - 1) % 2],
                    out_h.at[pl.ds(base + (c - 1) * chunk_rows, chunk_rows)],
                    semo))
        cpa[nch - 1].wait()
        cpb[nch - 1].wait()
        fma(nch - 1)
        cpo.append(pltpu.async_copy(
            bas[(nch - 1) % 2],
            out_h.at[pl.ds(base + (nch - 1) * chunk_rows, chunk_rows)],
            semo))
        cpo[nch - 2].wait()
        cpo[nch - 1].wait()

    return run(table, idx0, idx1, w0e)


def _mlp_body(be_ref, us_ref, xs_ref, gw_ref, pw_ref, ow_ref, ys_ref):
    b = pl.program_id(0)

    @pl.when(b < us_ref[0])
    def _compute():
        xb = xs_ref[...].astype(jnp.bfloat16)
        g = jnp.dot(xb, gw_ref[0], preferred_element_type=jnp.float32)
        p = jnp.dot(xb, pw_ref[0], preferred_element_type=jnp.float32)
        h = g * (p * jax.nn.sigmoid(p))
        ys_ref[...] = jnp.dot(h.astype(jnp.bfloat16), ow_ref[0],
                              preferred_element_type=jnp.float32)


def _run_mlp(xs, gwb, pwb, owb, block_expert, used):
    nslot, d = xs.shape
    fdim = gwb.shape[2]
    nblk = nslot // BLK
    grid_spec = pltpu.PrefetchScalarGridSpec(
        num_scalar_prefetch=2,
        grid=(nblk,),
        in_specs=[
            pl.BlockSpec((BLK, d),
                         lambda b, be, us: (jnp.minimum(b, us[0] - 1), 0)),
            pl.BlockSpec((1, d, fdim), lambda b, be, us: (be[b], 0, 0)),
            pl.BlockSpec((1, d, fdim), lambda b, be, us: (be[b], 0, 0)),
            pl.BlockSpec((1, fdim, d), lambda b, be, us: (be[b], 0, 0)),
        ],
        out_specs=pl.BlockSpec(
            (BLK, d), lambda b, be, us: (jnp.minimum(b, us[0] - 1), 0)),
    )
    return pl.pallas_call(
        _mlp_body,
        grid_spec=grid_spec,
        out_shape=jax.ShapeDtypeStruct((nslot, d), jnp.float32),
    )(block_expert, used, xs, gwb, pwb, owb)


def kernel(hidden_states, Wg, gw, pw, ow):
    b, s, d = hidden_states.shape
    x = hidden_states.reshape(-1, d).astype(jnp.float32)
    t = x.shape[0]
    ne, _, fdim = gw.shape
    nslot = TOPK * t + ne * BLK

    (logits, i0, i1, w0e, w1e, c0, call, _psum, bl) = _run_router(x, Wg)

    c0v = c0[0, :ne].astype(jnp.int32)
    callv = call[0, :ne].astype(jnp.int32)
    padded = ((callv + (BLK - 1)) // BLK) * BLK
    base = jnp.cumsum(padded) - padded
    off0 = jnp.zeros((1, 128), jnp.float32).at[0, :ne].set(base.astype(jnp.float32))
    off1 = jnp.zeros((1, 128), jnp.float32).at[0, :ne].set(
        (base + c0v).astype(jnp.float32))
    ends = (base + padded) // BLK
    nblk = nslot // BLK
    block_expert = jnp.minimum(
        jnp.sum(jnp.arange(nblk)[:, None] >= ends[None, :], axis=1),
        ne - 1).astype(jnp.int32)
    used = (jnp.sum(padded) // BLK).astype(jnp.int32).reshape(1)

    p0, p1 = _run_rank(i0, i1, off0, off1)
    p0f, p1f = p0.reshape(t), p1.reshape(t)

    ch = 16
    xs = _run_dispatch(x, p0f.reshape(t // ch, ch), p1f.reshape(t // ch, ch),
                       nslot, chunk_rows=ch)
    ys = _run_mlp(xs, gw.astype(jnp.bfloat16), pw.astype(jnp.bfloat16),
                  ow.astype(jnp.bfloat16), block_expert, used)
    out = _run_gather_combine(ys, p0f, p1f, w0e)

    return (out.reshape(b, s, d), logits, bl[0, 0])
